# baseline (speedup 1.0000x reference)
import functools

import numpy as np
import jax
import jax.numpy as jnp
from jax import lax
from jax.experimental import pallas as pl
from jax.experimental.pallas import tpu as pltpu
from jax.experimental.pallas import tpu_sc as plsc

_N = 524288
_D = 4
_NLEV = 4
_OUTD = _NLEV * _D
_BUCKETS = 1 << 22
_MASK = _BUCKETS - 1
_P0, _P1, _P2 = 73856093, 19349669, 83492791
_SCALES = (64.0, 128.0, 256.0, 512.0)
_IX = (0, 1, 0, 1, 0, 1, 0, 1)
_IY = (0, 0, 1, 1, 0, 0, 1, 1)
_IZ = (0, 0, 0, 0, 1, 1, 1, 1)
_CJ = tuple(
    int(np.uint32((_IX[j] * _P0 + _IY[j] * _P1 + _IZ[j] * _P2) & 0xFFFFFFFF)
        .astype(np.int32))
    for j in range(8)
)

_NC = 2
_NS = 16
_NW = _NC * _NS
_NT = _N // _NW
_B = 256
_NBLK = _NT // _B


def _body(pts_hbm, w0, w1, w2, w3, out_hbm,
          pts_v, frac_v, idx_v, par_v, rows_v, out_v, sem):
    tables = (w0, w1, w2, w3)
    i32 = jnp.int32
    cid = lax.axis_index("c").astype(i32)
    sid = lax.axis_index("s").astype(i32)
    wid = sid * i32(_NC) + cid
    tbase = wid * i32(_NT)
    iota = lax.iota(i32, 16)
    col0 = jnp.zeros(16, jnp.int32)
    col1 = jnp.full((16,), 1, jnp.int32)
    col2 = jnp.full((16,), 2, jnp.int32)

    def block(t, carry):
        base = tbase + t * i32(_B)
        pltpu.sync_copy(pts_hbm.at[pl.ds(base, _B)], pts_v)

        def phase_a(g, c):
            gi = g * i32(16) + iota
            x = plsc.load_gather(pts_v, [gi, col0])
            y = plsc.load_gather(pts_v, [gi, col1])
            z = plsc.load_gather(pts_v, [gi, col2])
            for lev in range(_NLEV):
                s = jnp.float32(_SCALES[lev])
                qx = x * s
                qy = y * s
                qz = z * s
                ix = qx.astype(jnp.int32)
                iy = qy.astype(jnp.int32)
                iz = qz.astype(jnp.int32)
                ix = ix - (ix.astype(jnp.float32) > qx).astype(jnp.int32)
                iy = iy - (iy.astype(jnp.float32) > qy).astype(jnp.int32)
                iz = iz - (iz.astype(jnp.float32) > qz).astype(jnp.int32)
                g16 = g * i32(16)
                frac_v[i32(lev), 0, pl.ds(g16, 16)] = qx - ix.astype(jnp.float32)
                frac_v[i32(lev), 1, pl.ds(g16, 16)] = qy - iy.astype(jnp.float32)
                frac_v[i32(lev), 2, pl.ds(g16, 16)] = qz - iz.astype(jnp.float32)
                h = (ix * jnp.int32(_P0) + iy * jnp.int32(_P1)
                     + iz * jnp.int32(_P2))
                ghi = g16 // i32(128)
                gcol = g16 % i32(128)
                for j in range(8):
                    vid = (h + jnp.int32(_CJ[j])) & jnp.int32(_MASK)
                    idx_v[i32(lev * 16 + j * 2) + ghi, pl.ds(gcol, 16)] = (
                        vid >> i32(1))
                    par_v[i32(lev), j, pl.ds(g16, 16)] = (
                        (vid & i32(1)) * i32(4))
            return c

        lax.fori_loop(i32(0), i32(_B // 16), phase_a, i32(0))

        handles = [
            pltpu.async_copy(
                tables[lev].at[idx_v.at[i32(lev * 16 + c)]],
                rows_v.at[i32(lev), pl.ds(i32(c * 128), 128)],
                sem)
            for lev in range(_NLEV)
            for c in range(16)
        ]
        for h in handles:
            h.wait()

        def phase_b(g, c):
            g16 = g * i32(16)
            gi = g16 + iota
            for lev in range(_NLEV):
                fx = frac_v[i32(lev), 0, pl.ds(g16, 16)]
                fy = frac_v[i32(lev), 1, pl.ds(g16, 16)]
                fz = frac_v[i32(lev), 2, pl.ds(g16, 16)]
                ux = (1.0 - fx, fx)
                uy = (1.0 - fy, fy)
                uz = (1.0 - fz, fz)
                w = [ux[_IX[j]] * uy[_IY[j]] * uz[_IZ[j]] for j in range(8)]
                rlev = rows_v.at[i32(lev)]
                par = [par_v[i32(lev), j, pl.ds(g16, 16)] for j in range(8)]
                for dd in range(_D):
                    acc = w[0] * plsc.load_gather(
                        rlev, [gi, par[0] + i32(dd)])
                    for j in range(1, 8):
                        acc = acc + w[j] * plsc.load_gather(
                            rlev, [jnp.int32(j * _B) + gi, par[j] + i32(dd)])
                    plsc.store_scatter(
                        out_v, [gi, jnp.full((16,), lev * _D + dd, jnp.int32)],
                        acc)
            return c

        lax.fori_loop(i32(0), i32(_B // 16), phase_b, i32(0))

        pltpu.sync_copy(out_v, out_hbm.at[pl.ds(base, _B)])
        return carry

    lax.fori_loop(i32(0), i32(_NBLK), block, i32(0))


_vox = functools.partial(
    pl.kernel,
    out_type=jax.ShapeDtypeStruct((_N, _OUTD), jnp.float32),
    mesh=plsc.VectorSubcoreMesh(
        core_axis_name="c", subcore_axis_name="s",
        num_cores=_NC, num_subcores=_NS),
    compiler_params=pltpu.CompilerParams(
        needs_layout_passes=False, use_tc_tiling_on_sc=False),
    scratch_types=[
        pltpu.VMEM((_B, 3), jnp.float32),
        pltpu.VMEM((_NLEV, 3, _B), jnp.float32),
        pltpu.VMEM((_NLEV * 8 * _B // 128, 128), jnp.int32),
        pltpu.VMEM((_NLEV, 8, _B), jnp.int32),
        pltpu.VMEM((_NLEV, 8 * _B, 2 * _D), jnp.float32),
        pltpu.VMEM((_B, _OUTD), jnp.float32),
        pltpu.SemaphoreType.DMA,
    ],
)(_body)


def kernel(pts, W0, W1, W2, W3):
    half = _BUCKETS // 2
    return _vox(pts,
                W0.reshape(half, 2 * _D), W1.reshape(half, 2 * _D),
                W2.reshape(half, 2 * _D), W3.reshape(half, 2 * _D))


# speedup vs baseline: 6.2832x; 6.2832x over previous
import functools

import numpy as np
import jax
import jax.numpy as jnp
from jax import lax
from jax.experimental import pallas as pl
from jax.experimental.pallas import tpu as pltpu
from jax.experimental.pallas import tpu_sc as plsc

_N = 524288
_D = 4
_NLEV = 4
_OUTD = _NLEV * _D
_BUCKETS = 1 << 22
_MASK = _BUCKETS - 1
_P0, _P1, _P2 = 73856093, 19349669, 83492791
_SCALES = (64.0, 128.0, 256.0, 512.0)
_IX = (0, 1, 0, 1, 0, 1, 0, 1)
_IY = (0, 0, 1, 1, 0, 0, 1, 1)
_IZ = (0, 0, 0, 0, 1, 1, 1, 1)
_CJ = tuple(
    int(np.uint32((_IX[j] * _P0 + _IY[j] * _P1 + _IZ[j] * _P2) & 0xFFFFFFFF)
        .astype(np.int32))
    for j in range(8)
)

_NC = 2
_NS = 16
_NW = _NC * _NS
_NT = _N // _NW
_B = 256
_NBLK = _NT // _B


_CH = 2048


def _tc_interleave_body(a0, a1, a2, a3, o0, o1, o2, o3):
    l = lax.broadcasted_iota(jnp.int32, (128, 128), 1)
    l2 = lax.broadcasted_iota(jnp.int32, (128, 128), 0)
    sels = [[((l % 4) == c) & (l2 == (32 * u + l // 4)) for c in range(4)]
            for u in range(4)]
    for a_ref, o_ref in ((a0, o0), (a1, o1), (a2, o2), (a3, o3)):
        a3d = a_ref[...].reshape(4, _CH // 128, 128)
        outs = []
        for u in range(4):
            acc = None
            for c in range(4):
                t = jnp.dot(a3d[c], sels[u][c].astype(jnp.float32),
                            preferred_element_type=jnp.float32)
                acc = t if acc is None else acc + t
            outs.append(acc)
        o_ref[...] = jnp.stack(outs, axis=1).reshape(_CH * 4 // 128, 128)


def _tc_interleave(w0t, w1t, w2t, w3t):
    n_out = _BUCKETS * _D // 128
    blk = _CH * 4 // 128
    return pl.pallas_call(
        _tc_interleave_body,
        grid=(_BUCKETS // _CH,),
        in_specs=[pl.BlockSpec((4, _CH), lambda g: (g * 0, g))] * 4,
        out_specs=[pl.BlockSpec((blk, 128), lambda g: (g, g * 0))] * 4,
        out_shape=[jax.ShapeDtypeStruct((n_out, 128), jnp.float32)] * 4,
    )(w0t, w1t, w2t, w3t)


def _body(pts_hbm, w0, w1, w2, w3, out_hbm,
          pts_v, frac_v, idx_v, par_v, rows_v, out_v, sem):
    tables = (w0, w1, w2, w3)
    i32 = jnp.int32
    cid = lax.axis_index("c").astype(i32)
    sid = lax.axis_index("s").astype(i32)
    wid = sid * i32(_NC) + cid
    tbase = wid * i32(_NT)
    iota = lax.iota(i32, 16)
    col0 = jnp.zeros(16, jnp.int32)
    col1 = jnp.full((16,), 1, jnp.int32)
    col2 = jnp.full((16,), 2, jnp.int32)

    def block(t, carry):
        base = tbase + t * i32(_B)
        pltpu.sync_copy(pts_hbm.at[pl.ds(base, _B)], pts_v)

        def phase_a(g, c):
            gi = g * i32(16) + iota
            x = plsc.load_gather(pts_v, [gi, col0])
            y = plsc.load_gather(pts_v, [gi, col1])
            z = plsc.load_gather(pts_v, [gi, col2])
            for lev in range(_NLEV):
                s = jnp.float32(_SCALES[lev])
                qx = x * s
                qy = y * s
                qz = z * s
                ix = qx.astype(jnp.int32)
                iy = qy.astype(jnp.int32)
                iz = qz.astype(jnp.int32)
                ix = ix - (ix.astype(jnp.float32) > qx).astype(jnp.int32)
                iy = iy - (iy.astype(jnp.float32) > qy).astype(jnp.int32)
                iz = iz - (iz.astype(jnp.float32) > qz).astype(jnp.int32)
                g16 = g * i32(16)
                frac_v[i32(lev), 0, pl.ds(g16, 16)] = qx - ix.astype(jnp.float32)
                frac_v[i32(lev), 1, pl.ds(g16, 16)] = qy - iy.astype(jnp.float32)
                frac_v[i32(lev), 2, pl.ds(g16, 16)] = qz - iz.astype(jnp.float32)
                h = (ix * jnp.int32(_P0) + iy * jnp.int32(_P1)
                     + iz * jnp.int32(_P2))
                ghi = g16 // i32(128)
                gcol = g16 % i32(128)
                for j in range(8):
                    vid = (h + jnp.int32(_CJ[j])) & jnp.int32(_MASK)
                    idx_v[i32(lev * 16 + j * 2) + ghi, pl.ds(gcol, 16)] = (
                        vid >> i32(1))
                    par_v[i32(lev), j, pl.ds(g16, 16)] = (
                        (vid & i32(1)) * i32(4))
            return c

        lax.fori_loop(i32(0), i32(_B // 16), phase_a, i32(0))

        handles = [
            pltpu.async_copy(
                tables[lev].at[idx_v.at[i32(lev * 16 + c)]],
                rows_v.at[i32(lev), pl.ds(i32(c * 128), 128)],
                sem)
            for lev in range(_NLEV)
            for c in range(16)
        ]
        for h in handles:
            h.wait()

        def phase_b(g, c):
            g16 = g * i32(16)
            gi = g16 + iota
            for lev in range(_NLEV):
                fx = frac_v[i32(lev), 0, pl.ds(g16, 16)]
                fy = frac_v[i32(lev), 1, pl.ds(g16, 16)]
                fz = frac_v[i32(lev), 2, pl.ds(g16, 16)]
                ux = (1.0 - fx, fx)
                uy = (1.0 - fy, fy)
                uz = (1.0 - fz, fz)
                w = [ux[_IX[j]] * uy[_IY[j]] * uz[_IZ[j]] for j in range(8)]
                rlev = rows_v.at[i32(lev)]
                par = [par_v[i32(lev), j, pl.ds(g16, 16)] for j in range(8)]
                jloc = g16 // i32(128)
                gcol = g16 % i32(128)
                for dd in range(_D):
                    acc = w[0] * plsc.load_gather(
                        rlev, [gi, par[0] + i32(dd)])
                    for j in range(1, 8):
                        acc = acc + w[j] * plsc.load_gather(
                            rlev, [jnp.int32(j * _B) + gi, par[j] + i32(dd)])
                    cc = lev * _D + dd
                    out_v[i32(cc // 8),
                          pl.ds(jloc * i32(1024) + i32((cc % 8) * 128)
                                + gcol, 16)] = acc
            return c

        lax.fori_loop(i32(0), i32(_B // 16), phase_b, i32(0))

        for i in range(2):
            pltpu.sync_copy(
                out_v.at[i32(i)],
                out_hbm.at[pl.ds(i32(i * (_N // 128) * 1024)
                                 + base * i32(8), 2048)])
        return carry

    lax.fori_loop(i32(0), i32(_NBLK), block, i32(0))


_vox = functools.partial(
    pl.kernel,
    out_type=jax.ShapeDtypeStruct((_N * _OUTD,), jnp.float32),
    mesh=plsc.VectorSubcoreMesh(
        core_axis_name="c", subcore_axis_name="s",
        num_cores=_NC, num_subcores=_NS),
    compiler_params=pltpu.CompilerParams(
        needs_layout_passes=False, use_tc_tiling_on_sc=False),
    scratch_types=[
        pltpu.VMEM((_B, 3), jnp.float32),
        pltpu.VMEM((_NLEV, 3, _B), jnp.float32),
        pltpu.VMEM((_NLEV * 8 * _B // 128, 128), jnp.int32),
        pltpu.VMEM((_NLEV, 8, _B), jnp.int32),
        pltpu.VMEM((_NLEV, 8 * _B, 2 * _D), jnp.float32),
        pltpu.VMEM((2, 2048), jnp.float32),
        pltpu.SemaphoreType.DMA,
    ],
)(_body)


def kernel(pts, W0, W1, W2, W3):
    ys = _tc_interleave(W0.T, W1.T, W2.T, W3.T)
    zs = [y.reshape(_BUCKETS // 2, 2 * _D) for y in ys]
    flat = _vox(pts, *zs)
    return (flat.reshape(2, _N // 128, 8, 128)
            .transpose(1, 3, 0, 2).reshape(_N, _OUTD))


# speedup vs baseline: 8.9585x; 1.4258x over previous
import functools

import numpy as np
import jax
import jax.numpy as jnp
from jax import lax
from jax.experimental import pallas as pl
from jax.experimental.pallas import tpu as pltpu
from jax.experimental.pallas import tpu_sc as plsc

_N = 524288
_D = 4
_NLEV = 4
_OUTD = _NLEV * _D
_BUCKETS = 1 << 22
_MASK = _BUCKETS - 1
_P0, _P1, _P2 = 73856093, 19349669, 83492791
_SCALES = (64.0, 128.0, 256.0, 512.0)
_IX = (0, 1, 0, 1, 0, 1, 0, 1)
_IY = (0, 0, 1, 1, 0, 0, 1, 1)
_IZ = (0, 0, 0, 0, 1, 1, 1, 1)
_CJ = tuple(
    int(np.uint32((_IX[j] * _P0 + _IY[j] * _P1 + _IZ[j] * _P2) & 0xFFFFFFFF)
        .astype(np.int32))
    for j in range(8)
)

_NC = 2
_NS = 16
_NW = _NC * _NS
_NT = _N // _NW
_B = 256
_NBLK = _NT // _B


_CH = 16384
_CHP = 16384


def _tc_interleave_body(a0, a1, a2, a3, o0, o1, o2, o3):
    l = lax.broadcasted_iota(jnp.int32, (128, 128), 1)
    l2 = lax.broadcasted_iota(jnp.int32, (128, 128), 0)
    sels = [[((l % 4) == c) & (l2 == (32 * u + l // 4)) for c in range(4)]
            for u in range(4)]
    for a_ref, o_ref in ((a0, o0), (a1, o1), (a2, o2), (a3, o3)):
        a3d = a_ref[...].reshape(4, _CH // 128, 128)
        outs = []
        for u in range(4):
            acc = None
            for c in range(4):
                t = jnp.dot(a3d[c], sels[u][c].astype(jnp.float32),
                            preferred_element_type=jnp.float32,
                            precision=lax.Precision.HIGHEST)
                acc = t if acc is None else acc + t
            outs.append(acc)
        o_ref[...] = jnp.stack(outs, axis=1).reshape(_CH * 4 // 128, 128)


def _tc_interleave(w0t, w1t, w2t, w3t):
    n_out = _BUCKETS * _D // 128
    blk = _CH * 4 // 128
    return pl.pallas_call(
        _tc_interleave_body,
        grid=(_BUCKETS // _CH,),
        in_specs=[pl.BlockSpec((4, _CH), lambda g: (g * 0, g))] * 4,
        out_specs=[pl.BlockSpec((blk, 128), lambda g: (g, g * 0))] * 4,
        out_shape=[jax.ShapeDtypeStruct((n_out, 128), jnp.float32)] * 4,
    )(w0t, w1t, w2t, w3t)


def _tc_pts_body(a_ref, o_ref):
    l = lax.broadcasted_iota(jnp.int32, (128, 128), 1)
    l2 = lax.broadcasted_iota(jnp.int32, (128, 128), 0)
    a3d = a_ref[...].reshape(3, _CHP // 128, 128)
    outs = []
    for p in range(3):
        acc = None
        for c in range(3):
            flat = p * 128 + l
            sel = ((flat % 3) == c) & (l2 == ((flat // 3) % 128))
            t = jnp.dot(a3d[c], sel.astype(jnp.float32),
                        preferred_element_type=jnp.float32,
                        precision=lax.Precision.HIGHEST)
            acc = t if acc is None else acc + t
        outs.append(acc)
    o_ref[...] = jnp.stack(outs, axis=1).reshape(_CHP * 3 // 128, 128)


def _tc_pts(ptst):
    blk = _CHP * 3 // 128
    return pl.pallas_call(
        _tc_pts_body,
        grid=(_N // _CHP,),
        in_specs=[pl.BlockSpec((3, _CHP), lambda g: (g * 0, g))],
        out_specs=pl.BlockSpec((blk, 128), lambda g: (g, g * 0)),
        out_shape=jax.ShapeDtypeStruct((_N * 3 // 128, 128), jnp.float32),
    )(ptst)


def _body(pts_hbm, w0, w1, w2, w3, out_hbm,
          pts_v, frac_v, idx_v, par_v, rows_v, out_v, sem):
    tables = (w0, w1, w2, w3)
    i32 = jnp.int32
    cid = lax.axis_index("c").astype(i32)
    sid = lax.axis_index("s").astype(i32)
    wid = sid * i32(_NC) + cid
    tbase = wid * i32(_NT)
    iota = lax.iota(i32, 16)
    iota3 = iota * i32(3)

    def block(t, carry):
        base = tbase + t * i32(_B)
        pltpu.sync_copy(pts_hbm.at[pl.ds(base * i32(3), 3 * _B)], pts_v)

        def phase_a(g, c):
            g16 = g * i32(16)
            gi3 = g16 * i32(3) + iota3
            x = plsc.load_gather(pts_v, [gi3])
            y = plsc.load_gather(pts_v, [gi3 + i32(1)])
            z = plsc.load_gather(pts_v, [gi3 + i32(2)])
            for lev in range(_NLEV):
                s = jnp.float32(_SCALES[lev])
                qx = x * s
                qy = y * s
                qz = z * s
                ix = qx.astype(jnp.int32)
                iy = qy.astype(jnp.int32)
                iz = qz.astype(jnp.int32)
                ix = ix - (ix.astype(jnp.float32) > qx).astype(jnp.int32)
                iy = iy - (iy.astype(jnp.float32) > qy).astype(jnp.int32)
                iz = iz - (iz.astype(jnp.float32) > qz).astype(jnp.int32)
                frac_v[i32(lev), 0, pl.ds(g16, 16)] = qx - ix.astype(jnp.float32)
                frac_v[i32(lev), 1, pl.ds(g16, 16)] = qy - iy.astype(jnp.float32)
                frac_v[i32(lev), 2, pl.ds(g16, 16)] = qz - iz.astype(jnp.float32)
                h = (ix * jnp.int32(_P0) + iy * jnp.int32(_P1)
                     + iz * jnp.int32(_P2))
                ghi = g16 // i32(128)
                gcol = g16 % i32(128)
                for j in range(8):
                    vid = (h + jnp.int32(_CJ[j])) & jnp.int32(_MASK)
                    idx_v[i32(lev * 16 + j * 2) + ghi, pl.ds(gcol, 16)] = (
                        vid >> i32(1))
                    par_v[i32(lev), j, pl.ds(g16, 16)] = (
                        (vid & i32(1)) * i32(4))
            return c

        lax.fori_loop(i32(0), i32(_B // 16), phase_a, i32(0))

        handles = [
            pltpu.async_copy(
                tables[lev].at[idx_v.at[i32(lev * 16 + c)]],
                rows_v.at[i32(lev), pl.ds(i32(c * 128), 128)],
                sem)
            for lev in range(_NLEV)
            for c in range(16)
        ]
        for h in handles:
            h.wait()

        def phase_b(g, c):
            g16 = g * i32(16)
            gi = g16 + iota
            for lev in range(_NLEV):
                fx = frac_v[i32(lev), 0, pl.ds(g16, 16)]
                fy = frac_v[i32(lev), 1, pl.ds(g16, 16)]
                fz = frac_v[i32(lev), 2, pl.ds(g16, 16)]
                ux = (1.0 - fx, fx)
                uy = (1.0 - fy, fy)
                uz = (1.0 - fz, fz)
                w = [ux[_IX[j]] * uy[_IY[j]] * uz[_IZ[j]] for j in range(8)]
                rlev = rows_v.at[i32(lev)]
                par = [par_v[i32(lev), j, pl.ds(g16, 16)] for j in range(8)]
                jloc = g16 // i32(128)
                gcol = g16 % i32(128)
                for dd in range(_D):
                    acc = w[0] * plsc.load_gather(
                        rlev, [gi, par[0] + i32(dd)])
                    for j in range(1, 8):
                        acc = acc + w[j] * plsc.load_gather(
                            rlev, [jnp.int32(j * _B) + gi, par[j] + i32(dd)])
                    cc = lev * _D + dd
                    out_v[i32(cc // 8),
                          pl.ds(jloc * i32(1024) + i32((cc % 8) * 128)
                                + gcol, 16)] = acc
            return c

        lax.fori_loop(i32(0), i32(_B // 16), phase_b, i32(0))

        for i in range(2):
            pltpu.sync_copy(
                out_v.at[i32(i)],
                out_hbm.at[pl.ds(i32(i * (_N // 128) * 1024)
                                 + base * i32(8), 2048)])
        return carry

    lax.fori_loop(i32(0), i32(_NBLK), block, i32(0))


_vox = functools.partial(
    pl.kernel,
    out_type=jax.ShapeDtypeStruct((_N * _OUTD,), jnp.float32),
    mesh=plsc.VectorSubcoreMesh(
        core_axis_name="c", subcore_axis_name="s",
        num_cores=_NC, num_subcores=_NS),
    compiler_params=pltpu.CompilerParams(
        needs_layout_passes=False, use_tc_tiling_on_sc=False),
    scratch_types=[
        pltpu.VMEM((3 * _B,), jnp.float32),
        pltpu.VMEM((_NLEV, 3, _B), jnp.float32),
        pltpu.VMEM((_NLEV * 8 * _B // 128, 128), jnp.int32),
        pltpu.VMEM((_NLEV, 8, _B), jnp.int32),
        pltpu.VMEM((_NLEV, 8 * _B, 2 * _D), jnp.float32),
        pltpu.VMEM((2, 2048), jnp.float32),
        pltpu.SemaphoreType.DMA,
    ],
)(_body)


def kernel(pts, W0, W1, W2, W3):
    ys = _tc_interleave(W0.T, W1.T, W2.T, W3.T)
    zs = [y.reshape(_BUCKETS // 2, 2 * _D) for y in ys]
    pts_lin = _tc_pts(pts.T).reshape(_N * 3)
    flat = _vox(pts_lin, *zs)
    return (flat.reshape(2, _N // 128, 8, 128)
            .transpose(1, 3, 0, 2).reshape(_N, _OUTD))


# speedup vs baseline: 11.2799x; 1.2591x over previous
import functools

import numpy as np
import jax
import jax.numpy as jnp
from jax import lax
from jax.experimental import pallas as pl
from jax.experimental.pallas import tpu as pltpu
from jax.experimental.pallas import tpu_sc as plsc

_N = 524288
_D = 4
_NLEV = 4
_OUTD = _NLEV * _D
_BUCKETS = 1 << 22
_MASK = _BUCKETS - 1
_P0, _P1, _P2 = 73856093, 19349669, 83492791
_SCALES = (64.0, 128.0, 256.0, 512.0)
_IX = (0, 1, 0, 1, 0, 1, 0, 1)
_IY = (0, 0, 1, 1, 0, 0, 1, 1)
_IZ = (0, 0, 0, 0, 1, 1, 1, 1)
_CJ = tuple(
    int(np.uint32((_IX[j] * _P0 + _IY[j] * _P1 + _IZ[j] * _P2) & 0xFFFFFFFF)
        .astype(np.int32))
    for j in range(8)
)

_NC = 2
_NS = 16
_NW = _NC * _NS
_NT = _N // _NW
_B = 256
_NBLK = _NT // _B


_CH = 16384
_CHP = 16384


def _tc_interleave_body(a0, a1, a2, a3, o0, o1, o2, o3):
    l = lax.broadcasted_iota(jnp.int32, (512, 128), 1)
    l2 = lax.broadcasted_iota(jnp.int32, (512, 128), 0)
    rs = []
    for u in range(4):
        c = l2 // 128
        sel = ((l % 4) == c) & ((l2 % 128) == (32 * u + l // 4))
        rs.append(sel.astype(jnp.bfloat16))
    for a_ref, o_ref in ((a0, o0), (a1, o1), (a2, o2), (a3, o3)):
        a3d = a_ref[...].reshape(4, _CH // 128, 128)
        av = jnp.concatenate([a3d[c] for c in range(4)], axis=1)
        hi = av.astype(jnp.bfloat16)
        r1 = av - hi.astype(jnp.float32)
        mid = r1.astype(jnp.bfloat16)
        lo = (r1 - mid.astype(jnp.float32)).astype(jnp.bfloat16)
        outs = []
        for u in range(4):
            acc = None
            for part in (hi, mid, lo):
                t = jnp.dot(part, rs[u],
                            preferred_element_type=jnp.float32)
                acc = t if acc is None else acc + t
            outs.append(acc)
        o_ref[...] = jnp.stack(outs, axis=1).reshape(_CH * 4 // 128, 128)


def _tc_interleave(w0t, w1t, w2t, w3t):
    n_out = _BUCKETS * _D // 128
    blk = _CH * 4 // 128
    return pl.pallas_call(
        _tc_interleave_body,
        grid=(_BUCKETS // _CH,),
        in_specs=[pl.BlockSpec((4, _CH), lambda g: (g * 0, g))] * 4,
        out_specs=[pl.BlockSpec((blk, 128), lambda g: (g, g * 0))] * 4,
        out_shape=[jax.ShapeDtypeStruct((n_out, 128), jnp.float32)] * 4,
    )(w0t, w1t, w2t, w3t)


def _tc_pts_body(a_ref, o_ref):
    l = lax.broadcasted_iota(jnp.int32, (128, 128), 1)
    l2 = lax.broadcasted_iota(jnp.int32, (128, 128), 0)
    a3d = a_ref[...].reshape(3, _CHP // 128, 128)
    outs = []
    for p in range(3):
        acc = None
        for c in range(3):
            flat = p * 128 + l
            sel = ((flat % 3) == c) & (l2 == ((flat // 3) % 128))
            t = jnp.dot(a3d[c], sel.astype(jnp.float32),
                        preferred_element_type=jnp.float32,
                        precision=lax.Precision.HIGHEST)
            acc = t if acc is None else acc + t
        outs.append(acc)
    o_ref[...] = jnp.stack(outs, axis=1).reshape(_CHP * 3 // 128, 128)


def _tc_pts(ptst):
    blk = _CHP * 3 // 128
    return pl.pallas_call(
        _tc_pts_body,
        grid=(_N // _CHP,),
        in_specs=[pl.BlockSpec((3, _CHP), lambda g: (g * 0, g))],
        out_specs=pl.BlockSpec((blk, 128), lambda g: (g, g * 0)),
        out_shape=jax.ShapeDtypeStruct((_N * 3 // 128, 128), jnp.float32),
    )(ptst)


def _body(pts_hbm, w0, w1, w2, w3, out_hbm,
          pts_v, frac_v, idx_v, par_v, rows_v, out_v, sem):
    tables = (w0, w1, w2, w3)
    i32 = jnp.int32
    cid = lax.axis_index("c").astype(i32)
    sid = lax.axis_index("s").astype(i32)
    wid = sid * i32(_NC) + cid
    tbase = wid * i32(_NT)
    iota = lax.iota(i32, 16)
    iota3 = iota * i32(3)

    def block(t, carry):
        base = tbase + t * i32(_B)
        pltpu.sync_copy(pts_hbm.at[pl.ds(base * i32(3), 3 * _B)], pts_v)

        def phase_a(g, c):
            g16 = g * i32(16)
            gi3 = g16 * i32(3) + iota3
            x = plsc.load_gather(pts_v, [gi3])
            y = plsc.load_gather(pts_v, [gi3 + i32(1)])
            z = plsc.load_gather(pts_v, [gi3 + i32(2)])
            for lev in range(_NLEV):
                s = jnp.float32(_SCALES[lev])
                qx = x * s
                qy = y * s
                qz = z * s
                ix = qx.astype(jnp.int32)
                iy = qy.astype(jnp.int32)
                iz = qz.astype(jnp.int32)
                ix = ix - (ix.astype(jnp.float32) > qx).astype(jnp.int32)
                iy = iy - (iy.astype(jnp.float32) > qy).astype(jnp.int32)
                iz = iz - (iz.astype(jnp.float32) > qz).astype(jnp.int32)
                frac_v[i32(lev), 0, pl.ds(g16, 16)] = qx - ix.astype(jnp.float32)
                frac_v[i32(lev), 1, pl.ds(g16, 16)] = qy - iy.astype(jnp.float32)
                frac_v[i32(lev), 2, pl.ds(g16, 16)] = qz - iz.astype(jnp.float32)
                h = (ix * jnp.int32(_P0) + iy * jnp.int32(_P1)
                     + iz * jnp.int32(_P2))
                ghi = g16 // i32(128)
                gcol = g16 % i32(128)
                for j in range(8):
                    vid = (h + jnp.int32(_CJ[j])) & jnp.int32(_MASK)
                    idx_v[i32(lev * 16 + j * 2) + ghi, pl.ds(gcol, 16)] = (
                        vid >> i32(1))
                    par_v[i32(lev), j, pl.ds(g16, 16)] = (
                        (vid & i32(1)) * i32(4))
            return c

        lax.fori_loop(i32(0), i32(_B // 16), phase_a, i32(0))

        handles = [
            pltpu.async_copy(
                tables[lev].at[idx_v.at[i32(lev * 16 + c)]],
                rows_v.at[i32(lev), pl.ds(i32(c * 128), 128)],
                sem)
            for lev in range(_NLEV)
            for c in range(16)
        ]
        for h in handles:
            h.wait()

        def phase_b(g, c):
            g16 = g * i32(16)
            gi = g16 + iota
            for lev in range(_NLEV):
                fx = frac_v[i32(lev), 0, pl.ds(g16, 16)]
                fy = frac_v[i32(lev), 1, pl.ds(g16, 16)]
                fz = frac_v[i32(lev), 2, pl.ds(g16, 16)]
                ux = (1.0 - fx, fx)
                uy = (1.0 - fy, fy)
                uz = (1.0 - fz, fz)
                w = [ux[_IX[j]] * uy[_IY[j]] * uz[_IZ[j]] for j in range(8)]
                rlev = rows_v.at[i32(lev)]
                par = [par_v[i32(lev), j, pl.ds(g16, 16)] for j in range(8)]
                jloc = g16 // i32(128)
                gcol = g16 % i32(128)
                for dd in range(_D):
                    acc = w[0] * plsc.load_gather(
                        rlev, [gi, par[0] + i32(dd)])
                    for j in range(1, 8):
                        acc = acc + w[j] * plsc.load_gather(
                            rlev, [jnp.int32(j * _B) + gi, par[j] + i32(dd)])
                    cc = lev * _D + dd
                    out_v[i32(cc // 8),
                          pl.ds(jloc * i32(1024) + i32((cc % 8) * 128)
                                + gcol, 16)] = acc
            return c

        lax.fori_loop(i32(0), i32(_B // 16), phase_b, i32(0))

        for i in range(2):
            pltpu.sync_copy(
                out_v.at[i32(i)],
                out_hbm.at[pl.ds(i32(i * (_N // 128) * 1024)
                                 + base * i32(8), 2048)])
        return carry

    lax.fori_loop(i32(0), i32(_NBLK), block, i32(0))


_vox = functools.partial(
    pl.kernel,
    out_type=jax.ShapeDtypeStruct((_N * _OUTD,), jnp.float32),
    mesh=plsc.VectorSubcoreMesh(
        core_axis_name="c", subcore_axis_name="s",
        num_cores=_NC, num_subcores=_NS),
    compiler_params=pltpu.CompilerParams(
        needs_layout_passes=False, use_tc_tiling_on_sc=False),
    scratch_types=[
        pltpu.VMEM((3 * _B,), jnp.float32),
        pltpu.VMEM((_NLEV, 3, _B), jnp.float32),
        pltpu.VMEM((_NLEV * 8 * _B // 128, 128), jnp.int32),
        pltpu.VMEM((_NLEV, 8, _B), jnp.int32),
        pltpu.VMEM((_NLEV, 8 * _B, 2 * _D), jnp.float32),
        pltpu.VMEM((2, 2048), jnp.float32),
        pltpu.SemaphoreType.DMA,
    ],
)(_body)


def kernel(pts, W0, W1, W2, W3):
    ys = _tc_interleave(W0.T, W1.T, W2.T, W3.T)
    zs = [y.reshape(_BUCKETS // 2, 2 * _D) for y in ys]
    pts_lin = _tc_pts(pts.T).reshape(_N * 3)
    flat = _vox(pts_lin, *zs)
    return (flat.reshape(2, _N // 128, 8, 128)
            .transpose(1, 3, 0, 2).reshape(_N, _OUTD))


# speedup vs baseline: 14.9415x; 1.3246x over previous
import functools

import numpy as np
import jax
import jax.numpy as jnp
from jax import lax
from jax.experimental import pallas as pl
from jax.experimental.pallas import tpu as pltpu
from jax.experimental.pallas import tpu_sc as plsc

_N = 524288
_D = 4
_NLEV = 4
_OUTD = _NLEV * _D
_BUCKETS = 1 << 22
_MASK = _BUCKETS - 1
_P0, _P1, _P2 = 73856093, 19349669, 83492791
_SCALES = (64.0, 128.0, 256.0, 512.0)
_IX = (0, 1, 0, 1, 0, 1, 0, 1)
_IY = (0, 0, 1, 1, 0, 0, 1, 1)
_IZ = (0, 0, 0, 0, 1, 1, 1, 1)
_CJ = tuple(
    int(np.uint32((_IX[j] * _P0 + _IY[j] * _P1 + _IZ[j] * _P2) & 0xFFFFFFFF)
        .astype(np.int32))
    for j in range(8)
)

_NC = 2
_NS = 16
_NW = _NC * _NS
_NT = _N // _NW
_B = 128
_NBLK = _NT // _B


_CH = 16384
_CHP = 16384


def _tc_interleave_body(a0, a1, a2, a3, o0, o1, o2, o3):
    l = lax.broadcasted_iota(jnp.int32, (512, 128), 1)
    l2 = lax.broadcasted_iota(jnp.int32, (512, 128), 0)
    rs = []
    for u in range(4):
        c = l2 // 128
        sel = ((l % 4) == c) & ((l2 % 128) == (32 * u + l // 4))
        rs.append(sel.astype(jnp.bfloat16))
    for a_ref, o_ref in ((a0, o0), (a1, o1), (a2, o2), (a3, o3)):
        a3d = a_ref[...].reshape(4, _CH // 128, 128)
        av = jnp.concatenate([a3d[c] for c in range(4)], axis=1)
        hi = av.astype(jnp.bfloat16)
        r1 = av - hi.astype(jnp.float32)
        mid = r1.astype(jnp.bfloat16)
        lo = (r1 - mid.astype(jnp.float32)).astype(jnp.bfloat16)
        outs = []
        for u in range(4):
            acc = None
            for part in (hi, mid, lo):
                t = jnp.dot(part, rs[u],
                            preferred_element_type=jnp.float32)
                acc = t if acc is None else acc + t
            outs.append(acc)
        o_ref[...] = jnp.stack(outs, axis=1).reshape(_CH * 4 // 128, 128)


def _tc_interleave(w0t, w1t, w2t, w3t):
    n_out = _BUCKETS * _D // 128
    blk = _CH * 4 // 128
    return pl.pallas_call(
        _tc_interleave_body,
        grid=(_BUCKETS // _CH,),
        in_specs=[pl.BlockSpec((4, _CH), lambda g: (g * 0, g))] * 4,
        out_specs=[pl.BlockSpec((blk, 128), lambda g: (g, g * 0))] * 4,
        out_shape=[jax.ShapeDtypeStruct((n_out, 128), jnp.float32)] * 4,
    )(w0t, w1t, w2t, w3t)


def _tc_pts_body(a_ref, o_ref):
    l = lax.broadcasted_iota(jnp.int32, (128, 128), 1)
    l2 = lax.broadcasted_iota(jnp.int32, (128, 128), 0)
    a3d = a_ref[...].reshape(3, _CHP // 128, 128)
    outs = []
    for p in range(3):
        acc = None
        for c in range(3):
            flat = p * 128 + l
            sel = ((flat % 3) == c) & (l2 == ((flat // 3) % 128))
            t = jnp.dot(a3d[c], sel.astype(jnp.float32),
                        preferred_element_type=jnp.float32,
                        precision=lax.Precision.HIGHEST)
            acc = t if acc is None else acc + t
        outs.append(acc)
    o_ref[...] = jnp.stack(outs, axis=1).reshape(_CHP * 3 // 128, 128)


def _tc_pts(ptst):
    blk = _CHP * 3 // 128
    return pl.pallas_call(
        _tc_pts_body,
        grid=(_N // _CHP,),
        in_specs=[pl.BlockSpec((3, _CHP), lambda g: (g * 0, g))],
        out_specs=pl.BlockSpec((blk, 128), lambda g: (g, g * 0)),
        out_shape=jax.ShapeDtypeStruct((_N * 3 // 128, 128), jnp.float32),
    )(ptst)


def _body(pts_hbm, w0, w1, w2, w3, out_hbm,
          pts_a, pts_b, frac_a, frac_b, idx_a, idx_b, par_a, par_b,
          rows_a, rows_b, out_a, out_b, sem_a, sem_b):
    tables = (w0, w1, w2, w3)
    i32 = jnp.int32
    cid = lax.axis_index("c").astype(i32)
    sid = lax.axis_index("s").astype(i32)
    wid = sid * i32(_NC) + cid
    tbase = wid * i32(_NT)
    iota = lax.iota(i32, 16)
    iota3 = iota * i32(3)
    buf0 = (pts_a, frac_a, idx_a, par_a, rows_a, out_a, sem_a)
    buf1 = (pts_b, frac_b, idx_b, par_b, rows_b, out_b, sem_b)

    def phase_a(tb, buf):
        pts_v, frac_v, idx_v, par_v = buf[0], buf[1], buf[2], buf[3]
        base = tbase + tb * i32(_B)
        pltpu.sync_copy(pts_hbm.at[pl.ds(base * i32(3), 3 * _B)], pts_v)

        def ga(g, c):
            g16 = g * i32(16)
            gi3 = g16 * i32(3) + iota3
            x = plsc.load_gather(pts_v, [gi3])
            y = plsc.load_gather(pts_v, [gi3 + i32(1)])
            z = plsc.load_gather(pts_v, [gi3 + i32(2)])
            for lev in range(_NLEV):
                sc = jnp.float32(_SCALES[lev])
                qx = x * sc
                qy = y * sc
                qz = z * sc
                ix = qx.astype(jnp.int32)
                iy = qy.astype(jnp.int32)
                iz = qz.astype(jnp.int32)
                ix = ix - (ix.astype(jnp.float32) > qx).astype(jnp.int32)
                iy = iy - (iy.astype(jnp.float32) > qy).astype(jnp.int32)
                iz = iz - (iz.astype(jnp.float32) > qz).astype(jnp.int32)
                frac_v[i32(lev), 0, pl.ds(g16, 16)] = qx - ix.astype(jnp.float32)
                frac_v[i32(lev), 1, pl.ds(g16, 16)] = qy - iy.astype(jnp.float32)
                frac_v[i32(lev), 2, pl.ds(g16, 16)] = qz - iz.astype(jnp.float32)
                h = (ix * jnp.int32(_P0) + iy * jnp.int32(_P1)
                     + iz * jnp.int32(_P2))
                for j in range(8):
                    vid = (h + jnp.int32(_CJ[j])) & jnp.int32(_MASK)
                    idx_v[i32(lev * 8 + j), pl.ds(g16, 16)] = vid >> i32(1)
                    par_v[i32(lev), j, pl.ds(g16, 16)] = (vid & i32(1)) * i32(4)
            return c

        lax.fori_loop(i32(0), i32(_B // 16), ga, i32(0))

    def fire(buf):
        idx_v, rows_v, sem = buf[2], buf[4], buf[6]
        for lev in range(_NLEV):
            for j in range(8):
                pltpu.async_copy(
                    tables[lev].at[idx_v.at[i32(lev * 8 + j)]],
                    rows_v.at[i32(lev), pl.ds(i32(j * 128), 128)], sem)

    def drain(buf):
        idx_v, rows_v, sem = buf[2], buf[4], buf[6]
        for lev in range(_NLEV):
            for j in range(8):
                pltpu.make_async_copy(
                    tables[lev].at[idx_v.at[i32(lev * 8 + j)]],
                    rows_v.at[i32(lev), pl.ds(i32(j * 128), 128)], sem).wait()

    def phase_b(tb, buf):
        frac_v, par_v, rows_v, out_v = buf[1], buf[3], buf[4], buf[5]
        base = tbase + tb * i32(_B)

        def gb(g, c):
            g16 = g * i32(16)
            gi = g16 + iota
            for lev in range(_NLEV):
                fx = frac_v[i32(lev), 0, pl.ds(g16, 16)]
                fy = frac_v[i32(lev), 1, pl.ds(g16, 16)]
                fz = frac_v[i32(lev), 2, pl.ds(g16, 16)]
                ux = (1.0 - fx, fx)
                uy = (1.0 - fy, fy)
                uz = (1.0 - fz, fz)
                w = [ux[_IX[j]] * uy[_IY[j]] * uz[_IZ[j]] for j in range(8)]
                rlev = rows_v.at[i32(lev)]
                par = [par_v[i32(lev), j, pl.ds(g16, 16)] for j in range(8)]
                for dd in range(_D):
                    acc = w[0] * plsc.load_gather(
                        rlev, [gi, par[0] + i32(dd)])
                    for j in range(1, 8):
                        acc = acc + w[j] * plsc.load_gather(
                            rlev, [jnp.int32(j * _B) + gi, par[j] + i32(dd)])
                    cc = lev * _D + dd
                    out_v[i32(cc // 8),
                          pl.ds(i32((cc % 8) * 128) + g16, 16)] = acc
            return c

        lax.fori_loop(i32(0), i32(_B // 16), gb, i32(0))
        for i in range(2):
            pltpu.sync_copy(
                out_v.at[i32(i)],
                out_hbm.at[pl.ds(i32(i * (_N // 128) * 1024)
                                 + base * i32(8), 1024)])

    phase_a(i32(0), buf0)
    fire(buf0)

    def pipe(t2, carry):
        b0 = t2 * i32(2)
        b1 = b0 + i32(1)
        phase_a(b1, buf1)
        fire(buf1)
        drain(buf0)
        phase_b(b0, buf0)

        @pl.when(t2 + i32(1) < i32(_NBLK // 2))
        def _():
            phase_a(b0 + i32(2), buf0)
            fire(buf0)

        drain(buf1)
        phase_b(b1, buf1)
        return carry

    lax.fori_loop(i32(0), i32(_NBLK // 2), pipe, i32(0))


_vox = functools.partial(
    pl.kernel,
    out_type=jax.ShapeDtypeStruct((_N * _OUTD,), jnp.float32),
    mesh=plsc.VectorSubcoreMesh(
        core_axis_name="c", subcore_axis_name="s",
        num_cores=_NC, num_subcores=_NS),
    compiler_params=pltpu.CompilerParams(
        needs_layout_passes=False, use_tc_tiling_on_sc=False),
    scratch_types=[
        pltpu.VMEM((3 * _B,), jnp.float32),
        pltpu.VMEM((3 * _B,), jnp.float32),
        pltpu.VMEM((_NLEV, 3, _B), jnp.float32),
        pltpu.VMEM((_NLEV, 3, _B), jnp.float32),
        pltpu.VMEM((_NLEV * 8, 128), jnp.int32),
        pltpu.VMEM((_NLEV * 8, 128), jnp.int32),
        pltpu.VMEM((_NLEV, 8, _B), jnp.int32),
        pltpu.VMEM((_NLEV, 8, _B), jnp.int32),
        pltpu.VMEM((_NLEV, 8 * _B, 2 * _D), jnp.float32),
        pltpu.VMEM((_NLEV, 8 * _B, 2 * _D), jnp.float32),
        pltpu.VMEM((2, 1024), jnp.float32),
        pltpu.VMEM((2, 1024), jnp.float32),
        pltpu.SemaphoreType.DMA,
        pltpu.SemaphoreType.DMA,
    ],
)(_body)


def kernel(pts, W0, W1, W2, W3):
    ys = _tc_interleave(W0.T, W1.T, W2.T, W3.T)
    zs = [y.reshape(_BUCKETS // 2, 2 * _D) for y in ys]
    pts_lin = _tc_pts(pts.T).reshape(_N * 3)
    flat = _vox(pts_lin, *zs)
    return (flat.reshape(2, _N // 128, 8, 128)
            .transpose(1, 3, 0, 2).reshape(_N, _OUTD))


# speedup vs baseline: 17.6090x; 1.1785x over previous
import functools

import numpy as np
import jax
import jax.numpy as jnp
from jax import lax
from jax.experimental import pallas as pl
from jax.experimental.pallas import tpu as pltpu
from jax.experimental.pallas import tpu_sc as plsc

_N = 524288
_D = 4
_NLEV = 4
_OUTD = _NLEV * _D
_BUCKETS = 1 << 22
_MASK = _BUCKETS - 1
_P0, _P1, _P2 = 73856093, 19349669, 83492791
_SCALES = (64.0, 128.0, 256.0, 512.0)
_IX = (0, 1, 0, 1, 0, 1, 0, 1)
_IY = (0, 0, 1, 1, 0, 0, 1, 1)
_IZ = (0, 0, 0, 0, 1, 1, 1, 1)
_CJ = tuple(
    int(np.uint32((_IX[j] * _P0 + _IY[j] * _P1 + _IZ[j] * _P2) & 0xFFFFFFFF)
        .astype(np.int32))
    for j in range(8)
)

_NC = 2
_NS = 16
_NW = _NC * _NS
_NT = _N // _NW
_B = 128
_NBLK = _NT // _B


_CH = 32768
_CHP = 16384


def _tc_interleave_body(a0, a1, a2, a3, o0, o1, o2, o3):
    m = _CH // 128
    l2 = lax.broadcasted_iota(jnp.int32, (512, 512), 0)
    n = lax.broadcasted_iota(jnp.int32, (512, 512), 1)
    u = n // 128
    l = n % 128
    r_all = (((l % 4) == l2 // 128)
             & ((l2 % 128) == (32 * u + l // 4))).astype(jnp.bfloat16)
    a_refs = (a0, a1, a2, a3)
    avs = []
    for t in range(4):
        a3d = a_refs[t][...].reshape(4, m, 128)
        avs.append(jnp.concatenate([a3d[c] for c in range(4)], axis=1))
    av_all = jnp.concatenate(avs, axis=0)
    hi = av_all.astype(jnp.bfloat16)
    r1 = av_all - hi.astype(jnp.float32)
    mid = r1.astype(jnp.bfloat16)
    lo = (r1 - mid.astype(jnp.float32)).astype(jnp.bfloat16)
    res = (jnp.dot(hi, r_all, preferred_element_type=jnp.float32)
           + jnp.dot(mid, r_all, preferred_element_type=jnp.float32)
           + jnp.dot(lo, r_all, preferred_element_type=jnp.float32))
    for t, o_ref in enumerate((o0, o1, o2, o3)):
        rt = res[t * m:(t + 1) * m]
        o_ref[...] = jnp.stack(
            [rt[:, uu * 128:(uu + 1) * 128] for uu in range(4)],
            axis=1).reshape(4 * m, 128)


def _tc_interleave(w0t, w1t, w2t, w3t):
    n_out = _BUCKETS * _D // 128
    blk = _CH * 4 // 128
    return pl.pallas_call(
        _tc_interleave_body,
        grid=(_BUCKETS // _CH,),
        in_specs=[pl.BlockSpec((4, _CH), lambda g: (g * 0, g))] * 4,
        out_specs=[pl.BlockSpec((blk, 128), lambda g: (g, g * 0))] * 4,
        out_shape=[jax.ShapeDtypeStruct((n_out, 128), jnp.float32)] * 4,
    )(w0t, w1t, w2t, w3t)


def _tc_pts_body(a_ref, o_ref):
    l = lax.broadcasted_iota(jnp.int32, (128, 128), 1)
    l2 = lax.broadcasted_iota(jnp.int32, (128, 128), 0)
    a3d = a_ref[...].reshape(3, _CHP // 128, 128)
    outs = []
    for p in range(3):
        acc = None
        for c in range(3):
            flat = p * 128 + l
            sel = ((flat % 3) == c) & (l2 == ((flat // 3) % 128))
            t = jnp.dot(a3d[c], sel.astype(jnp.float32),
                        preferred_element_type=jnp.float32,
                        precision=lax.Precision.HIGHEST)
            acc = t if acc is None else acc + t
        outs.append(acc)
    o_ref[...] = jnp.stack(outs, axis=1).reshape(_CHP * 3 // 128, 128)


def _tc_pts(ptst):
    blk = _CHP * 3 // 128
    return pl.pallas_call(
        _tc_pts_body,
        grid=(_N // _CHP,),
        in_specs=[pl.BlockSpec((3, _CHP), lambda g: (g * 0, g))],
        out_specs=pl.BlockSpec((blk, 128), lambda g: (g, g * 0)),
        out_shape=jax.ShapeDtypeStruct((_N * 3 // 128, 128), jnp.float32),
    )(ptst)


def _body(pts_hbm, w0, w1, w2, w3, out_hbm,
          pts_a, pts_b, frac_a, frac_b, idx_a, idx_b, par_a, par_b,
          rows_a, rows_b, out_a, out_b, sem_a, sem_b):
    tables = (w0, w1, w2, w3)
    i32 = jnp.int32
    cid = lax.axis_index("c").astype(i32)
    sid = lax.axis_index("s").astype(i32)
    wid = sid * i32(_NC) + cid
    tbase = wid * i32(_NT)
    iota = lax.iota(i32, 16)
    iota3 = iota * i32(3)
    buf0 = (pts_a, frac_a, idx_a, par_a, rows_a, out_a, sem_a)
    buf1 = (pts_b, frac_b, idx_b, par_b, rows_b, out_b, sem_b)

    def phase_a(tb, buf):
        pts_v, frac_v, idx_v, par_v = buf[0], buf[1], buf[2], buf[3]
        base = tbase + tb * i32(_B)
        pltpu.sync_copy(pts_hbm.at[pl.ds(base * i32(3), 3 * _B)], pts_v)

        def ga(g, c):
            g16 = g * i32(16)
            gi3 = g16 * i32(3) + iota3
            x = plsc.load_gather(pts_v, [gi3])
            y = plsc.load_gather(pts_v, [gi3 + i32(1)])
            z = plsc.load_gather(pts_v, [gi3 + i32(2)])
            for lev in range(_NLEV):
                sc = jnp.float32(_SCALES[lev])
                qx = x * sc
                qy = y * sc
                qz = z * sc
                ix = qx.astype(jnp.int32)
                iy = qy.astype(jnp.int32)
                iz = qz.astype(jnp.int32)
                ix = ix - (ix.astype(jnp.float32) > qx).astype(jnp.int32)
                iy = iy - (iy.astype(jnp.float32) > qy).astype(jnp.int32)
                iz = iz - (iz.astype(jnp.float32) > qz).astype(jnp.int32)
                frac_v[i32(lev), 0, pl.ds(g16, 16)] = qx - ix.astype(jnp.float32)
                frac_v[i32(lev), 1, pl.ds(g16, 16)] = qy - iy.astype(jnp.float32)
                frac_v[i32(lev), 2, pl.ds(g16, 16)] = qz - iz.astype(jnp.float32)
                h = (ix * jnp.int32(_P0) + iy * jnp.int32(_P1)
                     + iz * jnp.int32(_P2))
                for j in range(8):
                    vid = (h + jnp.int32(_CJ[j])) & jnp.int32(_MASK)
                    idx_v[i32(lev * 8 + j), pl.ds(g16, 16)] = vid >> i32(1)
                    par_v[i32(lev), j, pl.ds(g16, 16)] = (vid & i32(1)) * i32(4)
            return c

        lax.fori_loop(i32(0), i32(_B // 16), ga, i32(0))

    def fire(buf):
        idx_v, rows_v, sem = buf[2], buf[4], buf[6]
        for lev in range(_NLEV):
            for j in range(8):
                pltpu.async_copy(
                    tables[lev].at[idx_v.at[i32(lev * 8 + j)]],
                    rows_v.at[i32(lev), pl.ds(i32(j * 128), 128)], sem)

    def drain(buf):
        idx_v, rows_v, sem = buf[2], buf[4], buf[6]
        for lev in range(_NLEV):
            for j in range(8):
                pltpu.make_async_copy(
                    tables[lev].at[idx_v.at[i32(lev * 8 + j)]],
                    rows_v.at[i32(lev), pl.ds(i32(j * 128), 128)], sem).wait()

    def phase_b(tb, buf):
        frac_v, par_v, rows_v, out_v = buf[1], buf[3], buf[4], buf[5]
        base = tbase + tb * i32(_B)

        def gb(g, c):
            g16 = g * i32(16)
            gi = g16 + iota
            for lev in range(_NLEV):
                fx = frac_v[i32(lev), 0, pl.ds(g16, 16)]
                fy = frac_v[i32(lev), 1, pl.ds(g16, 16)]
                fz = frac_v[i32(lev), 2, pl.ds(g16, 16)]
                ux = (1.0 - fx, fx)
                uy = (1.0 - fy, fy)
                uz = (1.0 - fz, fz)
                w = [ux[_IX[j]] * uy[_IY[j]] * uz[_IZ[j]] for j in range(8)]
                rlev = rows_v.at[i32(lev)]
                par = [par_v[i32(lev), j, pl.ds(g16, 16)] for j in range(8)]
                for dd in range(_D):
                    acc = w[0] * plsc.load_gather(
                        rlev, [gi, par[0] + i32(dd)])
                    for j in range(1, 8):
                        acc = acc + w[j] * plsc.load_gather(
                            rlev, [jnp.int32(j * _B) + gi, par[j] + i32(dd)])
                    cc = lev * _D + dd
                    out_v[i32(cc // 8),
                          pl.ds(i32((cc % 8) * 128) + g16, 16)] = acc
            return c

        lax.fori_loop(i32(0), i32(_B // 16), gb, i32(0))
        for i in range(2):
            pltpu.sync_copy(
                out_v.at[i32(i)],
                out_hbm.at[pl.ds(i32(i * (_N // 128) * 1024)
                                 + base * i32(8), 1024)])

    phase_a(i32(0), buf0)
    fire(buf0)

    def pipe(t2, carry):
        b0 = t2 * i32(2)
        b1 = b0 + i32(1)
        phase_a(b1, buf1)
        fire(buf1)
        drain(buf0)
        phase_b(b0, buf0)

        @pl.when(t2 + i32(1) < i32(_NBLK // 2))
        def _():
            phase_a(b0 + i32(2), buf0)
            fire(buf0)

        drain(buf1)
        phase_b(b1, buf1)
        return carry

    lax.fori_loop(i32(0), i32(_NBLK // 2), pipe, i32(0))


_vox = functools.partial(
    pl.kernel,
    out_type=jax.ShapeDtypeStruct((_N * _OUTD,), jnp.float32),
    mesh=plsc.VectorSubcoreMesh(
        core_axis_name="c", subcore_axis_name="s",
        num_cores=_NC, num_subcores=_NS),
    compiler_params=pltpu.CompilerParams(
        needs_layout_passes=False, use_tc_tiling_on_sc=False),
    scratch_types=[
        pltpu.VMEM((3 * _B,), jnp.float32),
        pltpu.VMEM((3 * _B,), jnp.float32),
        pltpu.VMEM((_NLEV, 3, _B), jnp.float32),
        pltpu.VMEM((_NLEV, 3, _B), jnp.float32),
        pltpu.VMEM((_NLEV * 8, 128), jnp.int32),
        pltpu.VMEM((_NLEV * 8, 128), jnp.int32),
        pltpu.VMEM((_NLEV, 8, _B), jnp.int32),
        pltpu.VMEM((_NLEV, 8, _B), jnp.int32),
        pltpu.VMEM((_NLEV, 8 * _B, 2 * _D), jnp.float32),
        pltpu.VMEM((_NLEV, 8 * _B, 2 * _D), jnp.float32),
        pltpu.VMEM((2, 1024), jnp.float32),
        pltpu.VMEM((2, 1024), jnp.float32),
        pltpu.SemaphoreType.DMA,
        pltpu.SemaphoreType.DMA,
    ],
)(_body)


def kernel(pts, W0, W1, W2, W3):
    ys = _tc_interleave(W0.T, W1.T, W2.T, W3.T)
    zs = [y.reshape(_BUCKETS // 2, 2 * _D) for y in ys]
    pts_lin = _tc_pts(pts.T).reshape(_N * 3)
    flat = _vox(pts_lin, *zs)
    return (flat.reshape(2, _N // 128, 8, 128)
            .transpose(1, 3, 0, 2).reshape(_N, _OUTD))


# speedup vs baseline: 19.3699x; 1.1000x over previous
import functools

import numpy as np
import jax
import jax.numpy as jnp
from jax import lax
from jax.experimental import pallas as pl
from jax.experimental.pallas import tpu as pltpu
from jax.experimental.pallas import tpu_sc as plsc

_N = 524288
_D = 4
_NLEV = 4
_OUTD = _NLEV * _D
_BUCKETS = 1 << 22
_MASK = _BUCKETS - 1
_P0, _P1, _P2 = 73856093, 19349669, 83492791
_SCALES = (64.0, 128.0, 256.0, 512.0)
_IX = (0, 1, 0, 1, 0, 1, 0, 1)
_IY = (0, 0, 1, 1, 0, 0, 1, 1)
_IZ = (0, 0, 0, 0, 1, 1, 1, 1)
_CJ = tuple(
    int(np.uint32((_IX[j] * _P0 + _IY[j] * _P1 + _IZ[j] * _P2) & 0xFFFFFFFF)
        .astype(np.int32))
    for j in range(8)
)

_NC = 2
_NS = 16
_NW = _NC * _NS
_NT = _N // _NW
_B = 128
_NBLK = _NT // _B


_CH = 65536
_CHP = 16384


def _tc_interleave_body(a0, a1, a2, a3, o0, o1, o2, o3):
    m = _CH // 128
    l2 = lax.broadcasted_iota(jnp.int32, (512, 512), 0)
    n = lax.broadcasted_iota(jnp.int32, (512, 512), 1)
    u = n // 128
    l = n % 128
    r_all = (((l % 4) == l2 // 128)
             & ((l2 % 128) == (32 * u + l // 4))).astype(jnp.bfloat16)
    a_refs = (a0, a1, a2, a3)
    avs = []
    for t in range(4):
        a3d = a_refs[t][...].reshape(4, m, 128)
        avs.append(jnp.concatenate([a3d[c] for c in range(4)], axis=1))
    av_all = jnp.concatenate(avs, axis=0)
    hi = av_all.astype(jnp.bfloat16)
    mid = (av_all - hi.astype(jnp.float32)).astype(jnp.bfloat16)
    res = (jnp.dot(hi, r_all, preferred_element_type=jnp.float32)
           + jnp.dot(mid, r_all, preferred_element_type=jnp.float32))
    for t, o_ref in enumerate((o0, o1, o2, o3)):
        rt = res[t * m:(t + 1) * m]
        o_ref[...] = jnp.stack(
            [rt[:, uu * 128:(uu + 1) * 128] for uu in range(4)],
            axis=1).reshape(4 * m, 128)


def _tc_interleave(w0t, w1t, w2t, w3t):
    n_out = _BUCKETS * _D // 128
    blk = _CH * 4 // 128
    return pl.pallas_call(
        _tc_interleave_body,
        grid=(_BUCKETS // _CH,),
        in_specs=[pl.BlockSpec((4, _CH), lambda g: (g * 0, g))] * 4,
        out_specs=[pl.BlockSpec((blk, 128), lambda g: (g, g * 0))] * 4,
        out_shape=[jax.ShapeDtypeStruct((n_out, 128), jnp.float32)] * 4,
    )(w0t, w1t, w2t, w3t)


def _tc_pts_body(a_ref, o_ref):
    l = lax.broadcasted_iota(jnp.int32, (128, 128), 1)
    l2 = lax.broadcasted_iota(jnp.int32, (128, 128), 0)
    a3d = a_ref[...].reshape(3, _CHP // 128, 128)
    outs = []
    for p in range(3):
        acc = None
        for c in range(3):
            flat = p * 128 + l
            sel = ((flat % 3) == c) & (l2 == ((flat // 3) % 128))
            t = jnp.dot(a3d[c], sel.astype(jnp.float32),
                        preferred_element_type=jnp.float32,
                        precision=lax.Precision.HIGHEST)
            acc = t if acc is None else acc + t
        outs.append(acc)
    o_ref[...] = jnp.stack(outs, axis=1).reshape(_CHP * 3 // 128, 128)


def _tc_pts(ptst):
    blk = _CHP * 3 // 128
    return pl.pallas_call(
        _tc_pts_body,
        grid=(_N // _CHP,),
        in_specs=[pl.BlockSpec((3, _CHP), lambda g: (g * 0, g))],
        out_specs=pl.BlockSpec((blk, 128), lambda g: (g, g * 0)),
        out_shape=jax.ShapeDtypeStruct((_N * 3 // 128, 128), jnp.float32),
    )(ptst)


def _body(pts_hbm, w0, w1, w2, w3, out_hbm,
          pts_v, out_v, frac_a, frac_b, frac_c, idx_a, idx_b, idx_c,
          par_a, par_b, par_c, rows_a, rows_b, rows_c,
          sem_a, sem_b, sem_c):
    tables = (w0, w1, w2, w3)
    i32 = jnp.int32
    cid = lax.axis_index("c").astype(i32)
    sid = lax.axis_index("s").astype(i32)
    wid = sid * i32(_NC) + cid
    tbase = wid * i32(_NT)
    iota = lax.iota(i32, 16)
    iota3 = iota * i32(3)
    bufs = ((frac_a, idx_a, par_a, rows_a, sem_a),
            (frac_b, idx_b, par_b, rows_b, sem_b),
            (frac_c, idx_c, par_c, rows_c, sem_c))

    def phase_a(tb, buf):
        frac_v, idx_v, par_v = buf[0], buf[1], buf[2]
        base = tbase + tb * i32(_B)
        pltpu.sync_copy(pts_hbm.at[pl.ds(base * i32(3), 3 * _B)], pts_v)

        def ga(g, c):
            g16 = g * i32(16)
            gi3 = g16 * i32(3) + iota3
            x = plsc.load_gather(pts_v, [gi3])
            y = plsc.load_gather(pts_v, [gi3 + i32(1)])
            z = plsc.load_gather(pts_v, [gi3 + i32(2)])
            for lev in range(_NLEV):
                sc = jnp.float32(_SCALES[lev])
                qx = x * sc
                qy = y * sc
                qz = z * sc
                ix = qx.astype(jnp.int32)
                iy = qy.astype(jnp.int32)
                iz = qz.astype(jnp.int32)
                ix = ix - (ix.astype(jnp.float32) > qx).astype(jnp.int32)
                iy = iy - (iy.astype(jnp.float32) > qy).astype(jnp.int32)
                iz = iz - (iz.astype(jnp.float32) > qz).astype(jnp.int32)
                frac_v[i32(lev), 0, pl.ds(g16, 16)] = qx - ix.astype(jnp.float32)
                frac_v[i32(lev), 1, pl.ds(g16, 16)] = qy - iy.astype(jnp.float32)
                frac_v[i32(lev), 2, pl.ds(g16, 16)] = qz - iz.astype(jnp.float32)
                h = (ix * jnp.int32(_P0) + iy * jnp.int32(_P1)
                     + iz * jnp.int32(_P2))
                for j in range(8):
                    vid = (h + jnp.int32(_CJ[j])) & jnp.int32(_MASK)
                    idx_v[i32(lev * 8 + j), pl.ds(g16, 16)] = vid >> i32(1)
                    par_v[i32(lev), j, pl.ds(g16, 16)] = (vid & i32(1)) * i32(4)
            return c

        lax.fori_loop(i32(0), i32(_B // 16), ga, i32(0))

    def fire(buf):
        idx_v, rows_v, sem = buf[1], buf[3], buf[4]
        for lev in range(_NLEV):
            for j in range(8):
                pltpu.async_copy(
                    tables[lev].at[idx_v.at[i32(lev * 8 + j)]],
                    rows_v.at[i32(lev), pl.ds(i32(j * 128), 128)], sem)

    def drain(buf):
        idx_v, rows_v, sem = buf[1], buf[3], buf[4]
        for lev in range(_NLEV):
            for j in range(8):
                pltpu.make_async_copy(
                    tables[lev].at[idx_v.at[i32(lev * 8 + j)]],
                    rows_v.at[i32(lev), pl.ds(i32(j * 128), 128)], sem).wait()

    def phase_b(tb, buf):
        frac_v, par_v, rows_v = buf[0], buf[2], buf[3]
        base = tbase + tb * i32(_B)

        def gb(g, c):
            g16 = g * i32(16)
            gi = g16 + iota
            for lev in range(_NLEV):
                fx = frac_v[i32(lev), 0, pl.ds(g16, 16)]
                fy = frac_v[i32(lev), 1, pl.ds(g16, 16)]
                fz = frac_v[i32(lev), 2, pl.ds(g16, 16)]
                ux = (1.0 - fx, fx)
                uy = (1.0 - fy, fy)
                uz = (1.0 - fz, fz)
                txy = [uy[0] * ux[0], uy[0] * ux[1], uy[1] * ux[0],
                       uy[1] * ux[1]]
                w = [txy[2 * _IY[j] + _IX[j]] * uz[_IZ[j]] for j in range(8)]
                rlev = rows_v.at[i32(lev)]
                par = [par_v[i32(lev), j, pl.ds(g16, 16)] for j in range(8)]
                for dd in range(_D):
                    acc = w[0] * plsc.load_gather(
                        rlev, [gi, par[0] + i32(dd)])
                    for j in range(1, 8):
                        acc = acc + w[j] * plsc.load_gather(
                            rlev, [jnp.int32(j * _B) + gi, par[j] + i32(dd)])
                    cc = lev * _D + dd
                    out_v[i32(cc // 8),
                          pl.ds(i32((cc % 8) * 128) + g16, 16)] = acc
            return c

        lax.fori_loop(i32(0), i32(_B // 16), gb, i32(0))
        for i in range(2):
            pltpu.sync_copy(
                out_v.at[i32(i)],
                out_hbm.at[pl.ds(i32(i * (_N // 128) * 1024)
                                 + base * i32(8), 1024)])

    for r in range(3):
        phase_a(i32(r), bufs[r])
        fire(bufs[r])

    def pipe(k, carry):
        for r in range(3):
            b = k * i32(3) + i32(r)

            @pl.when(b < i32(_NBLK))
            def _():
                drain(bufs[r])
                phase_b(b, bufs[r])

            @pl.when(b + i32(3) < i32(_NBLK))
            def _():
                phase_a(b + i32(3), bufs[r])
                fire(bufs[r])
        return carry

    lax.fori_loop(i32(0), i32((_NBLK + 2) // 3), pipe, i32(0))


_vox = functools.partial(
    pl.kernel,
    out_type=jax.ShapeDtypeStruct((_N * _OUTD,), jnp.float32),
    mesh=plsc.VectorSubcoreMesh(
        core_axis_name="c", subcore_axis_name="s",
        num_cores=_NC, num_subcores=_NS),
    compiler_params=pltpu.CompilerParams(
        needs_layout_passes=False, use_tc_tiling_on_sc=False),
    scratch_types=[
        pltpu.VMEM((3 * _B,), jnp.float32),
        pltpu.VMEM((2, 1024), jnp.float32),
        pltpu.VMEM((_NLEV, 3, _B), jnp.float32),
        pltpu.VMEM((_NLEV, 3, _B), jnp.float32),
        pltpu.VMEM((_NLEV, 3, _B), jnp.float32),
        pltpu.VMEM((_NLEV * 8, 128), jnp.int32),
        pltpu.VMEM((_NLEV * 8, 128), jnp.int32),
        pltpu.VMEM((_NLEV * 8, 128), jnp.int32),
        pltpu.VMEM((_NLEV, 8, _B), jnp.int32),
        pltpu.VMEM((_NLEV, 8, _B), jnp.int32),
        pltpu.VMEM((_NLEV, 8, _B), jnp.int32),
        pltpu.VMEM((_NLEV, 8 * _B, 2 * _D), jnp.float32),
        pltpu.VMEM((_NLEV, 8 * _B, 2 * _D), jnp.float32),
        pltpu.VMEM((_NLEV, 8 * _B, 2 * _D), jnp.float32),
        pltpu.SemaphoreType.DMA,
        pltpu.SemaphoreType.DMA,
        pltpu.SemaphoreType.DMA,
    ],
)(_body)


def kernel(pts, W0, W1, W2, W3):
    ys = _tc_interleave(W0.T, W1.T, W2.T, W3.T)
    zs = [y.reshape(_BUCKETS // 2, 2 * _D) for y in ys]
    pts_lin = _tc_pts(pts.T).reshape(_N * 3)
    flat = _vox(pts_lin, *zs)
    return (flat.reshape(2, _N // 128, 8, 128)
            .transpose(1, 3, 0, 2).reshape(_N, _OUTD))
